# baseline (speedup 1.0000x reference)
import functools

import jax
import jax.numpy as jnp
from jax import lax
from jax.experimental import pallas as pl
from jax.experimental.pallas import tpu as pltpu
from jax.experimental.pallas import tpu_sc as plsc

_NC = 2
_NS = 16
_NW = _NC * _NS
_CHUNK = 128
_PADW = 128
_NBUF = 4
_L = 16


@functools.lru_cache(maxsize=None)
def _build_gather_t(n_t: int, emb_dim: int, batch: int):
    n_fr = emb_dim // 8
    n_bc = batch // _CHUNK
    assert n_bc == _NW and n_t % _NBUF == 0

    @functools.partial(
        pl.kernel,
        out_type=jax.ShapeDtypeStruct((n_t, n_fr, n_bc, 8, _CHUNK), jnp.float32),
        mesh=plsc.VectorSubcoreMesh(core_axis_name="c", subcore_axis_name="s"),
        scratch_types=[
            pltpu.VMEM((n_t, _CHUNK), jnp.int32),
            pltpu.VMEM((_NBUF, _CHUNK, emb_dim), jnp.float32),
            pltpu.VMEM((_NBUF, n_fr, 8, _CHUNK), jnp.float32),
            [pltpu.SemaphoreType.DMA] * _NBUF,
            [pltpu.SemaphoreType.DMA] * _NBUF,
        ],
        compiler_params=pltpu.CompilerParams(
            use_tc_tiling_on_sc=False, needs_layout_passes=False
        ),
    )
    def gather_kernel(idx_hbm, table_hbm, out_hbm, idx_v, rows_v, outb, gsems, osems):
        wid = lax.axis_index("s") * _NC + lax.axis_index("c")
        pltpu.sync_copy(idx_hbm.at[:, pl.ds(wid * _CHUNK, _CHUNK)], idx_v)

        row16 = [lax.iota(jnp.int32, _L) + j2 * _L for j2 in range(_CHUNK // _L)]

        @pl.loop(0, n_t, step=_NBUF)
        def _(t0):
            gathers = [
                pltpu.async_copy(
                    table_hbm.at[idx_v.at[t0 + j]], rows_v.at[j], gsems[j]
                )
                for j in range(_NBUF)
            ]
            for j in range(_NBUF):
                gathers[j].wait()

                @pl.when(t0 > 0)
                def _():
                    pltpu.make_async_copy(
                        out_hbm.at[0, :, 0], outb.at[j], osems[j]
                    ).wait()

                @pl.loop(0, n_fr)
                def _(fr):
                    for f2 in range(8):
                        colv = jnp.broadcast_to(fr * 8 + f2, (_L,)).astype(jnp.int32)
                        for j2 in range(_CHUNK // _L):
                            v = plsc.load_gather(rows_v.at[j], [row16[j2], colv])
                            outb[j, fr, f2, pl.ds(j2 * _L, _L)] = v

                pltpu.async_copy(
                    outb.at[j], out_hbm.at[t0 + j, :, wid], osems[j]
                )

        for j in range(_NBUF):
            pltpu.make_async_copy(out_hbm.at[0, :, 0], outb.at[j], osems[j]).wait()

    return gather_kernel


def kernel(item_ids, item_emb):
    batch, hist = item_ids.shape
    n_items, emb_dim = item_emb.shape
    assert batch % (_NW * _CHUNK // _CHUNK) == 0 and hist % _NBUF == 0
    n_tab = (n_items + 7) // 8 * 8
    halves = n_tab * _PADW // emb_dim
    table = jnp.pad(item_emb, ((0, n_tab - n_items), (0, _PADW - emb_dim)))
    table = table.reshape(halves, emb_dim)
    scale = _PADW // emb_dim
    ids_t = (item_ids.astype(jnp.int32) * scale).T
    out5 = _build_gather_t(hist, emb_dim, batch)(ids_t, table)
    out = out5.transpose(2, 4, 0, 1, 3).reshape(batch, hist, emb_dim)
    return out


# speedup vs baseline: 1.1843x; 1.1843x over previous
import functools

import jax
import jax.numpy as jnp
from jax import lax
from jax.experimental import pallas as pl
from jax.experimental.pallas import tpu as pltpu
from jax.experimental.pallas import tpu_sc as plsc

_NC = 2
_NS = 16
_NW = _NC * _NS
_CHUNK = 128
_PADW = 128
_NBUF = 4
_L = 16


@functools.lru_cache(maxsize=None)
def _build_gather_t(n_t: int, emb_dim: int, batch: int):
    n_fr = emb_dim // 8
    n_bc = batch // _CHUNK
    assert n_bc == _NW and n_t % _NBUF == 0

    @functools.partial(
        pl.kernel,
        out_type=jax.ShapeDtypeStruct((n_t, n_fr, n_bc, 8, _CHUNK), jnp.float32),
        mesh=plsc.VectorSubcoreMesh(core_axis_name="c", subcore_axis_name="s"),
        scratch_types=[
            pltpu.VMEM((n_t, _CHUNK), jnp.int32),
            pltpu.VMEM((_NBUF, _CHUNK, emb_dim), jnp.float32),
            pltpu.VMEM((_NBUF, n_fr, 8, _CHUNK), jnp.float32),
            [pltpu.SemaphoreType.DMA] * _NBUF,
            [pltpu.SemaphoreType.DMA] * _NBUF,
        ],
        compiler_params=pltpu.CompilerParams(
            use_tc_tiling_on_sc=False, needs_layout_passes=False
        ),
    )
    def gather_kernel(idx_hbm, table_hbm, out_hbm, idx_v, rows_v, outb, gsems, osems):
        wid = lax.axis_index("s") * _NC + lax.axis_index("c")
        pltpu.sync_copy(idx_hbm.at[:, pl.ds(wid * _CHUNK, _CHUNK)], idx_v)

        row16 = [lax.iota(jnp.int32, _L) + j2 * _L for j2 in range(_CHUNK // _L)]

        @pl.loop(0, n_t, step=_NBUF)
        def _(t0):
            gathers = [
                pltpu.async_copy(
                    table_hbm.at[idx_v.at[t0 + j]], rows_v.at[j], gsems[j]
                )
                for j in range(_NBUF)
            ]
            for j in range(_NBUF):
                gathers[j].wait()

                @pl.when(t0 > 0)
                def _():
                    pltpu.make_async_copy(
                        out_hbm.at[0, :, 0], outb.at[j], osems[j]
                    ).wait()

                @plsc.parallel_loop(0, n_fr, 1)
                def _(fr):
                    for f2 in range(0, 8, 2):
                        vs = []
                        for f2b in (f2, f2 + 1):
                            colv = jnp.broadcast_to(fr * 8 + f2b, (_L,)).astype(
                                jnp.int32
                            )
                            vs += [
                                plsc.load_gather(rows_v.at[j], [row16[j2], colv])
                                for j2 in range(_CHUNK // _L)
                            ]
                        for k, f2b in enumerate((f2, f2 + 1)):
                            for j2 in range(_CHUNK // _L):
                                outb[j, fr, f2b, pl.ds(j2 * _L, _L)] = vs[
                                    k * (_CHUNK // _L) + j2
                                ]

                pltpu.async_copy(
                    outb.at[j], out_hbm.at[t0 + j, :, wid], osems[j]
                )

        for j in range(_NBUF):
            pltpu.make_async_copy(out_hbm.at[0, :, 0], outb.at[j], osems[j]).wait()

    return gather_kernel


def kernel(item_ids, item_emb):
    batch, hist = item_ids.shape
    n_items, emb_dim = item_emb.shape
    assert batch % (_NW * _CHUNK // _CHUNK) == 0 and hist % _NBUF == 0
    n_tab = (n_items + 7) // 8 * 8
    halves = n_tab * _PADW // emb_dim
    table = jnp.pad(item_emb, ((0, n_tab - n_items), (0, _PADW - emb_dim)))
    table = table.reshape(halves, emb_dim)
    scale = _PADW // emb_dim
    ids_t = (item_ids.astype(jnp.int32) * scale).T
    out5 = _build_gather_t(hist, emb_dim, batch)(ids_t, table)
    out = out5.transpose(2, 4, 0, 1, 3).reshape(batch, hist, emb_dim)
    return out


# speedup vs baseline: 2.1243x; 1.7937x over previous
import functools

import jax
import jax.numpy as jnp
from jax import lax
from jax.experimental import pallas as pl
from jax.experimental.pallas import tpu as pltpu
from jax.experimental.pallas import tpu_sc as plsc

_NC = 2
_NS = 16
_NW = _NC * _NS
_CHUNK = 256
_PADW = 128
_NBUF = 4


@functools.lru_cache(maxsize=None)
def _build_gather(n_rows: int, emb_dim: int, n_chunks: int, n_tab: int):
    @functools.partial(
        pl.kernel,
        out_type=jax.ShapeDtypeStruct((n_rows, _PADW), jnp.float32),
        mesh=plsc.VectorSubcoreMesh(core_axis_name="c", subcore_axis_name="s"),
        scratch_types=[
            pltpu.VMEM((n_chunks, _CHUNK), jnp.int32),
            pltpu.VMEM((_NBUF, _CHUNK, emb_dim), jnp.float32),
            [pltpu.SemaphoreType.DMA] * _NBUF,
        ],
        compiler_params=pltpu.CompilerParams(use_tc_tiling_on_sc=False),
    )
    def gather_kernel(idx_hbm, table_hbm, out_hbm, idx_v, rows_v, sems):
        wid = lax.axis_index("s") * _NC + lax.axis_index("c")
        pltpu.sync_copy(idx_hbm.at[wid], idx_v)
        base = wid * (n_chunks * _CHUNK)

        @pl.loop(0, n_chunks, step=_NBUF)
        def _(c):
            copies = [
                pltpu.async_copy(
                    table_hbm.at[idx_v.at[c + j]], rows_v.at[j], sems[j]
                )
                for j in range(_NBUF)
            ]
            for j in range(_NBUF):
                copies[j].wait()
                pltpu.sync_copy(
                    rows_v.at[j],
                    out_hbm.at[
                        pl.ds(base + (c + j) * _CHUNK, _CHUNK), pl.ds(0, emb_dim)
                    ],
                )

    return gather_kernel


def kernel(item_ids, item_emb):
    batch, hist = item_ids.shape
    n_items, emb_dim = item_emb.shape
    n_rows = batch * hist
    assert n_rows % (_NW * _CHUNK) == 0
    n_chunks = n_rows // (_NW * _CHUNK)
    n_tab = (n_items + 7) // 8 * 8
    halves = n_tab * _PADW // emb_dim
    table = jnp.pad(item_emb, ((0, n_tab - n_items), (0, _PADW - emb_dim)))
    table = table.reshape(halves, emb_dim)
    scale = _PADW // emb_dim
    ids = (item_ids.astype(jnp.int32) * scale).reshape(_NW, n_chunks, _CHUNK)
    out = _build_gather(n_rows, emb_dim, n_chunks, n_tab)(ids, table)
    return out.reshape(batch, hist, _PADW)[:, :, :emb_dim]
